# baseline (speedup 1.0000x reference)
import functools

import jax
import jax.numpy as jnp
from jax import lax
from jax.experimental import pallas as pl
from jax.experimental.pallas import tpu as pltpu
from jax.experimental.pallas import tpu_sc as plsc

N = 10000
E = 320000
D = 128
N2 = 10000
A2 = 20000
N3 = 10000
A3 = 30000
NI2 = 32
NI3 = 64
NCLS = 10
B = 256

SC_CORES = 2
SC_TILES = 16
NW = SC_CORES * SC_TILES
CH = 128

NP = 10112
BP = 384
RB = 2528


def _round_up(a, m):
    return (a + m - 1) // m * m


@functools.cache
def _sc_segsum(n_src, w, e_pad, n_pad, kg):
    e_tile = e_pad // NW
    n_chunks = e_tile // CH
    ngrp = n_chunks // kg
    assert n_chunks % kg == 0 and ngrp % 2 == 0
    rpt = n_pad // SC_TILES

    mesh = plsc.VectorSubcoreMesh(
        core_axis_name="c", subcore_axis_name="s",
        num_cores=SC_CORES, num_subcores=SC_TILES)

    @functools.partial(
        pl.kernel,
        out_type=jax.ShapeDtypeStruct((SC_CORES, n_pad, w), jnp.float32),
        mesh=mesh,
        compiler_params=pltpu.CompilerParams(use_tc_tiling_on_sc=False),
        scratch_types=[
            pltpu.VMEM_SHARED((n_pad, w), jnp.float32),
            pltpu.VMEM((n_chunks, CH), jnp.int32),
            pltpu.VMEM((n_chunks, CH), jnp.int32),
            pltpu.VMEM((2, kg, CH, w), jnp.float32),
            pltpu.SemaphoreType.DMA,
            pltpu.SemaphoreType.DMA,
            pltpu.SemaphoreType.DMA,
        ],
    )
    def seg(table, src, dst, zeros, out, acc, idx_s, idx_d, rows,
            sem_g0, sem_g1, sem_s):
        cid = lax.axis_index("c")
        sid = lax.axis_index("s")
        wid = cid * SC_TILES + sid
        sems = (sem_g0, sem_g1)

        pltpu.sync_copy(zeros, acc.at[pl.ds(sid * rpt, rpt)])
        pltpu.sync_copy(src.at[pl.ds(wid * n_chunks, n_chunks)], idx_s)
        pltpu.sync_copy(dst.at[pl.ds(wid * n_chunks, n_chunks)], idx_d)
        plsc.subcore_barrier()

        def fire(g, s):
            for b in range(kg):
                pltpu.async_copy(table.at[idx_s.at[g * kg + b]],
                                 rows.at[s, b], sems[s])

        def drain_scatter(g, s):
            for b in range(kg):
                pltpu.make_async_copy(table.at[idx_s.at[0]],
                                      rows.at[s, b], sems[s]).wait()
            ds = [pltpu.async_copy(rows.at[s, b], acc.at[idx_d.at[g * kg + b]],
                                   sem_s, add=True) for b in range(kg)]
            for d in ds:
                d.wait()

        fire(0, 0)

        def grp2(h, carry):
            g0 = h * 2
            fire(g0 + 1, 1)
            drain_scatter(g0, 0)

            @pl.when(g0 + 2 < ngrp)
            def _():
                fire(g0 + 2, 0)

            drain_scatter(g0 + 1, 1)
            return carry

        lax.fori_loop(0, ngrp // 2, grp2, 0)
        plsc.subcore_barrier()
        pltpu.sync_copy(acc.at[pl.ds(sid * rpt, rpt)],
                        out.at[cid, pl.ds(sid * rpt, rpt)])

    return seg


def _pad_edges(src, dst, e_pad, dummy_dst):
    e = src.shape[0]
    src = jnp.pad(src, (0, e_pad - e)).reshape(-1, CH)
    dst = jnp.pad(dst, (0, e_pad - e),
                  constant_values=dummy_dst).reshape(-1, CH)
    return src, dst


def _elu(v):
    return jnp.where(v > 0.0, v, jnp.exp(jnp.minimum(v, 0.0)) - 1.0)


def _row_spec(w):
    return pl.BlockSpec((RB, w), lambda i: (i, 0))


def _pair_spec(w, which):
    return pl.BlockSpec((1, RB, w), lambda i, _w=which: (_w, i, 0))


def _full_spec():
    return pl.BlockSpec(lambda i: None)


def _wspec(shape):
    nd = len(shape)
    return pl.BlockSpec(shape, lambda i: (0,) * nd)


def _tc_call(body, in_specs, out_w, n_out=1):
    shp = jax.ShapeDtypeStruct((NP, out_w), jnp.float32)
    if n_out == 1:
        out_shape, out_specs = shp, _row_spec(out_w)
    else:
        out_shape = (shp,) * n_out
        out_specs = tuple(_row_spec(out_w) for _ in range(n_out))
    return pl.pallas_call(
        body, grid=(NP // RB,), in_specs=in_specs,
        out_specs=out_specs, out_shape=out_shape)


def _two_mm(x_ref, wr_ref, wt_ref, y_ref, t_ref):
    xv = x_ref[...]
    y_ref[...] = jnp.dot(xv, wr_ref[...], preferred_element_type=jnp.float32)
    t_ref[...] = jnp.dot(xv, wt_ref[...], preferred_element_type=jnp.float32)


def _combine_elu(s_ref0, s_ref1, t_ref, b_ref, o_ref):
    o_ref[...] = _elu(s_ref0[0] + s_ref1[0] + t_ref[...] + b_ref[...])


def _agg_first(s_ref0, s_ref1, h_ref, wr_ref, wt_ref, b_ref, o_ref, *, aug):
    a = s_ref0[0] + s_ref1[0]
    h = h_ref[...]
    v = _elu(jnp.dot(a, wr_ref[...], preferred_element_type=jnp.float32)
             + jnp.dot(h, wt_ref[...], preferred_element_type=jnp.float32)
             + b_ref[...])
    if aug:
        o_ref[...] = jnp.concatenate(
            [v, jnp.ones((v.shape[0], 16), jnp.float32)], axis=1)
    else:
        o_ref[...] = v


def _pool_transform(s_ref0, s_ref1, iso_ref, wrp_ref, wri_ref, wtp_ref,
                    wti_ref, y_ref, t_ref):
    s = s_ref0[0] + s_ref1[0]
    p = s[:, :64] / jnp.maximum(s[:, 64:65], 1.0)
    iso = iso_ref[...]
    f32 = jnp.float32
    y_ref[...] = (jnp.dot(p, wrp_ref[...], preferred_element_type=f32)
                  + jnp.dot(iso, wri_ref[...], preferred_element_type=f32))
    t_ref[...] = (jnp.dot(p, wtp_ref[...], preferred_element_type=f32)
                  + jnp.dot(iso, wti_ref[...], preferred_element_type=f32))


def _head(x1_ref, x2_ref, x3_ref, w1_ref, b1_ref, w2_ref, b2_ref,
          w3_ref, b3_ref, o_ref):
    def pool(ref):
        s = ref[0] + ref[1]
        return (s[:, :64] / jnp.maximum(s[:, 64:65], 1.0))[:B]

    z = jnp.concatenate([pool(x1_ref), pool(x2_ref), pool(x3_ref)], axis=1)
    f32 = jnp.float32
    z = _elu(jnp.dot(z, w1_ref[...], preferred_element_type=f32) + b1_ref[...])
    z = _elu(jnp.dot(z, w2_ref[...], preferred_element_type=f32) + b2_ref[...])
    z = jnp.dot(z, w3_ref[...], preferred_element_type=f32) + b3_ref[...]
    m = jnp.max(z, axis=1, keepdims=True)
    e = jnp.exp(z - m)
    o_ref[...] = z - m - jnp.log(jnp.sum(e, axis=1, keepdims=True))


def kernel(x, edge_index, batch, assignment_index_2, iso_type_2, edge_index_2,
           batch_2, assignment_index_3, iso_type_3, edge_index_3, batch_3,
           c1_wr, c1_br, c1_wt, c2_wr, c2_br, c2_wt, c3_wr, c3_br, c3_wt,
           c4_wr, c4_br, c4_wt, c5_wr, c5_br, c5_wt, c6_wr, c6_br, c6_wt,
           c7_wr, c7_br, c7_wt, fc1_w, fc1_b, fc2_w, fc2_b, fc3_w, fc3_b):
    f32 = jnp.float32
    e_pad = _round_up(E, NW * CH * 8)
    a2_pad = _round_up(A2, NW * CH * 4)
    a3_pad = _round_up(A3, NW * CH * 4)
    nb_pad = _round_up(N, NW * CH * 4)

    src1, dst1 = _pad_edges(edge_index[0], edge_index[1], e_pad, N)
    src2, dst2 = _pad_edges(edge_index_2[0], edge_index_2[1], e_pad, N2)
    src3, dst3 = _pad_edges(edge_index_3[0], edge_index_3[1], e_pad, N3)
    sa2, da2 = _pad_edges(assignment_index_2[0], assignment_index_2[1],
                          a2_pad, N2)
    sa3, da3 = _pad_edges(assignment_index_3[0], assignment_index_3[1],
                          a3_pad, N3)
    iota = jnp.arange(N, dtype=jnp.int32)
    sb1, db1 = _pad_edges(iota, batch, nb_pad, B)
    sb2, db2 = _pad_edges(iota, batch_2, nb_pad, B)
    sb3, db3 = _pad_edges(iota, batch_3, nb_pad, B)

    z32 = jnp.zeros((NP // SC_TILES, 32), f32)
    z64 = jnp.zeros((NP // SC_TILES, 64), f32)
    z80 = jnp.zeros((NP // SC_TILES, 80), f32)
    zb80 = jnp.zeros((BP // SC_TILES, 80), f32)

    x_p = jnp.pad(x, ((0, NP - N), (0, 0)))
    iso2_p = jnp.pad(iso_type_2, ((0, NP - N2), (0, 0)))
    iso3_p = jnp.pad(iso_type_3, ((0, NP - N3), (0, 0)))

    agg32 = _sc_segsum(NP, 32, e_pad, NP, 4)
    agg64 = _sc_segsum(NP, 64, e_pad, NP, 4)
    pool_a2 = _sc_segsum(NP, 80, a2_pad, NP, 2)
    pool_a3 = _sc_segsum(NP, 80, a3_pad, NP, 2)
    pool_b = _sc_segsum(NP, 80, nb_pad, BP, 2)

    y1, t1 = _tc_call(
        _two_mm, [_row_spec(D), _wspec((D, 32)), _wspec((D, 32))], 32, 2)(
            x_p, c1_wr, c1_wt)
    s1 = agg32(y1, src1, dst1, z32)
    h1 = _tc_call(
        _combine_elu,
        [_pair_spec(32, 0), _pair_spec(32, 1), _row_spec(32), _wspec((32,))],
        32)(s1, s1, t1, c1_br)

    s2 = agg32(h1, src1, dst1, z32)
    h2 = _tc_call(
        functools.partial(_agg_first, aug=False),
        [_pair_spec(32, 0), _pair_spec(32, 1), _row_spec(32),
         _wspec((32, 64)), _wspec((32, 64)), _wspec((64,))],
        64)(s2, s2, h1, c2_wr, c2_wt, c2_br)

    s3 = agg64(h2, src1, dst1, z64)
    h3a = _tc_call(
        functools.partial(_agg_first, aug=True),
        [_pair_spec(64, 0), _pair_spec(64, 1), _row_spec(64),
         _wspec((64, 64)), _wspec((64, 64)), _wspec((64,))],
        80)(s3, s3, h2, c3_wr, c3_wt, c3_br)

    x1s = pool_b(h3a, sb1, db1, zb80)
    p2s = pool_a2(h3a, sa2, da2, z80)
    p3s = pool_a3(h3a, sa3, da3, z80)

    y4, t4 = _tc_call(
        _pool_transform,
        [_pair_spec(80, 0), _pair_spec(80, 1), _row_spec(NI2),
         _wspec((64, 64)), _wspec((NI2, 64)), _wspec((64, 64)),
         _wspec((NI2, 64))],
        64, 2)(p2s, p2s, iso2_p, c4_wr[:64], c4_wr[64:], c4_wt[:64],
               c4_wt[64:])
    s4 = agg64(y4, src2, dst2, z64)
    g1 = _tc_call(
        _combine_elu,
        [_pair_spec(64, 0), _pair_spec(64, 1), _row_spec(64), _wspec((64,))],
        64)(s4, s4, t4, c4_br)

    s5 = agg64(g1, src2, dst2, z64)
    g2a = _tc_call(
        functools.partial(_agg_first, aug=True),
        [_pair_spec(64, 0), _pair_spec(64, 1), _row_spec(64),
         _wspec((64, 64)), _wspec((64, 64)), _wspec((64,))],
        80)(s5, s5, g1, c5_wr, c5_wt, c5_br)
    x2s = pool_b(g2a, sb2, db2, zb80)

    y6, t6 = _tc_call(
        _pool_transform,
        [_pair_spec(80, 0), _pair_spec(80, 1), _row_spec(NI3),
         _wspec((64, 64)), _wspec((NI3, 64)), _wspec((64, 64)),
         _wspec((NI3, 64))],
        64, 2)(p3s, p3s, iso3_p, c6_wr[:64], c6_wr[64:], c6_wt[:64],
               c6_wt[64:])
    s6 = agg64(y6, src3, dst3, z64)
    m1 = _tc_call(
        _combine_elu,
        [_pair_spec(64, 0), _pair_spec(64, 1), _row_spec(64), _wspec((64,))],
        64)(s6, s6, t6, c6_br)

    s7 = agg64(m1, src3, dst3, z64)
    m2a = _tc_call(
        functools.partial(_agg_first, aug=True),
        [_pair_spec(64, 0), _pair_spec(64, 1), _row_spec(64),
         _wspec((64, 64)), _wspec((64, 64)), _wspec((64,))],
        80)(s7, s7, m1, c7_wr, c7_wt, c7_br)
    x3s = pool_b(m2a, sb3, db3, zb80)

    out = pl.pallas_call(
        _head,
        out_shape=jax.ShapeDtypeStruct((B, NCLS), f32),
    )(x1s, x2s, x3s, fc1_w, fc1_b, fc2_w, fc2_b, fc3_w, fc3_b)
    return out


# speedup vs baseline: 3.2429x; 3.2429x over previous
import functools

import jax
import jax.numpy as jnp
from jax import lax
from jax.experimental import pallas as pl
from jax.experimental.pallas import tpu as pltpu
from jax.experimental.pallas import tpu_sc as plsc

N = 10000
E = 320000
D = 128
N2 = 10000
A2 = 20000
N3 = 10000
A3 = 30000
NI2 = 32
NI3 = 64
NCLS = 10
B = 256

SC_CORES = 2
SC_TILES = 16
NW = SC_CORES * SC_TILES
CH = 64

NP = 10112
BP = 384
RB = 2528


def _round_up(a, m):
    return (a + m - 1) // m * m


@functools.cache
def _sc_segsum(n_src, w, e_pad, n_pad, kg):
    e_tile = e_pad // NW
    n_chunks = e_tile // CH
    ngrp = n_chunks // kg
    assert n_chunks % kg == 0 and ngrp % 2 == 0
    rpt = n_pad // SC_TILES
    spt = n_src // SC_TILES

    mesh = plsc.VectorSubcoreMesh(
        core_axis_name="c", subcore_axis_name="s",
        num_cores=SC_CORES, num_subcores=SC_TILES)

    @functools.partial(
        pl.kernel,
        out_type=jax.ShapeDtypeStruct((SC_CORES, n_pad, w), jnp.float32),
        mesh=mesh,
        compiler_params=pltpu.CompilerParams(use_tc_tiling_on_sc=False),
        scratch_types=[
            pltpu.VMEM_SHARED((n_src, w), jnp.float32),
            pltpu.VMEM_SHARED((n_pad, w), jnp.float32),
            pltpu.VMEM((n_chunks, CH), jnp.int32),
            pltpu.VMEM((n_chunks, CH), jnp.int32),
            pltpu.VMEM((2, kg, CH, w), jnp.float32),
            pltpu.SemaphoreType.DMA,
            pltpu.SemaphoreType.DMA,
            pltpu.SemaphoreType.DMA,
        ],
    )
    def seg(table, src, dst, zeros, out, tbl, acc, idx_s, idx_d, rows,
            sem_g0, sem_g1, sem_s):
        cid = lax.axis_index("c")
        sid = lax.axis_index("s")
        wid = cid * SC_TILES + sid
        sems = (sem_g0, sem_g1)

        pltpu.sync_copy(table.at[pl.ds(sid * spt, spt)],
                        tbl.at[pl.ds(sid * spt, spt)])
        pltpu.sync_copy(zeros, acc.at[pl.ds(sid * rpt, rpt)])
        pltpu.sync_copy(src.at[pl.ds(wid * n_chunks, n_chunks)], idx_s)
        pltpu.sync_copy(dst.at[pl.ds(wid * n_chunks, n_chunks)], idx_d)
        plsc.subcore_barrier()

        def fire(g, s):
            for b in range(kg):
                pltpu.async_copy(tbl.at[idx_s.at[g * kg + b]],
                                 rows.at[s, b], sems[s])

        def drain_scatter(g, s):
            for b in range(kg):
                pltpu.make_async_copy(tbl.at[idx_s.at[0]],
                                      rows.at[s, b], sems[s]).wait()
            ds = [pltpu.async_copy(rows.at[s, b], acc.at[idx_d.at[g * kg + b]],
                                   sem_s, add=True) for b in range(kg)]
            for d in ds:
                d.wait()

        fire(0, 0)

        def grp2(h, carry):
            g0 = h * 2
            fire(g0 + 1, 1)
            drain_scatter(g0, 0)

            @pl.when(g0 + 2 < ngrp)
            def _():
                fire(g0 + 2, 0)

            drain_scatter(g0 + 1, 1)
            return carry

        lax.fori_loop(0, ngrp // 2, grp2, 0)
        plsc.subcore_barrier()
        pltpu.sync_copy(acc.at[pl.ds(sid * rpt, rpt)],
                        out.at[cid, pl.ds(sid * rpt, rpt)])

    return seg


def _pad_edges(src, dst, e_pad, dummy_dst):
    e = src.shape[0]
    src = jnp.pad(src, (0, e_pad - e)).reshape(-1, CH)
    dst = jnp.pad(dst, (0, e_pad - e),
                  constant_values=dummy_dst).reshape(-1, CH)
    return src, dst


def _elu(v):
    return jnp.where(v > 0.0, v, jnp.exp(jnp.minimum(v, 0.0)) - 1.0)


def _row_spec(w):
    return pl.BlockSpec((RB, w), lambda i: (i, 0))


def _pair_spec(w, which):
    return pl.BlockSpec((1, RB, w), lambda i, _w=which: (_w, i, 0))


def _full_spec():
    return pl.BlockSpec(lambda i: None)


def _wspec(shape):
    nd = len(shape)
    return pl.BlockSpec(shape, lambda i: (0,) * nd)


def _tc_call(body, in_specs, out_w, n_out=1):
    shp = jax.ShapeDtypeStruct((NP, out_w), jnp.float32)
    if n_out == 1:
        out_shape, out_specs = shp, _row_spec(out_w)
    else:
        out_shape = (shp,) * n_out
        out_specs = tuple(_row_spec(out_w) for _ in range(n_out))
    return pl.pallas_call(
        body, grid=(NP // RB,), in_specs=in_specs,
        out_specs=out_specs, out_shape=out_shape)


def _two_mm(x_ref, wr_ref, wt_ref, y_ref, t_ref):
    xv = x_ref[...]
    y_ref[...] = jnp.dot(xv, wr_ref[...], preferred_element_type=jnp.float32)
    t_ref[...] = jnp.dot(xv, wt_ref[...], preferred_element_type=jnp.float32)


def _combine_elu(s_ref0, s_ref1, t_ref, b_ref, o_ref):
    o_ref[...] = _elu(s_ref0[0] + s_ref1[0] + t_ref[...] + b_ref[...])


def _agg_first(s_ref0, s_ref1, h_ref, wr_ref, wt_ref, b_ref, o_ref, *, aug):
    a = s_ref0[0] + s_ref1[0]
    h = h_ref[...]
    v = _elu(jnp.dot(a, wr_ref[...], preferred_element_type=jnp.float32)
             + jnp.dot(h, wt_ref[...], preferred_element_type=jnp.float32)
             + b_ref[...])
    if aug:
        o_ref[...] = jnp.concatenate(
            [v, jnp.ones((v.shape[0], 16), jnp.float32)], axis=1)
    else:
        o_ref[...] = v


def _pool_transform(s_ref0, s_ref1, iso_ref, wrp_ref, wri_ref, wtp_ref,
                    wti_ref, y_ref, t_ref):
    s = s_ref0[0] + s_ref1[0]
    p = s[:, :64] / jnp.maximum(s[:, 64:65], 1.0)
    iso = iso_ref[...]
    f32 = jnp.float32
    y_ref[...] = (jnp.dot(p, wrp_ref[...], preferred_element_type=f32)
                  + jnp.dot(iso, wri_ref[...], preferred_element_type=f32))
    t_ref[...] = (jnp.dot(p, wtp_ref[...], preferred_element_type=f32)
                  + jnp.dot(iso, wti_ref[...], preferred_element_type=f32))


def _head(x1_ref, x2_ref, x3_ref, w1_ref, b1_ref, w2_ref, b2_ref,
          w3_ref, b3_ref, o_ref):
    def pool(ref):
        s = ref[0] + ref[1]
        return (s[:, :64] / jnp.maximum(s[:, 64:65], 1.0))[:B]

    z = jnp.concatenate([pool(x1_ref), pool(x2_ref), pool(x3_ref)], axis=1)
    f32 = jnp.float32
    z = _elu(jnp.dot(z, w1_ref[...], preferred_element_type=f32) + b1_ref[...])
    z = _elu(jnp.dot(z, w2_ref[...], preferred_element_type=f32) + b2_ref[...])
    z = jnp.dot(z, w3_ref[...], preferred_element_type=f32) + b3_ref[...]
    m = jnp.max(z, axis=1, keepdims=True)
    e = jnp.exp(z - m)
    o_ref[...] = z - m - jnp.log(jnp.sum(e, axis=1, keepdims=True))


def kernel(x, edge_index, batch, assignment_index_2, iso_type_2, edge_index_2,
           batch_2, assignment_index_3, iso_type_3, edge_index_3, batch_3,
           c1_wr, c1_br, c1_wt, c2_wr, c2_br, c2_wt, c3_wr, c3_br, c3_wt,
           c4_wr, c4_br, c4_wt, c5_wr, c5_br, c5_wt, c6_wr, c6_br, c6_wt,
           c7_wr, c7_br, c7_wt, fc1_w, fc1_b, fc2_w, fc2_b, fc3_w, fc3_b):
    f32 = jnp.float32
    e_pad = _round_up(E, NW * CH * 4)
    a2_pad = _round_up(A2, NW * CH * 4)
    a3_pad = _round_up(A3, NW * CH * 4)
    nb_pad = _round_up(N, NW * CH * 4)

    src1, dst1 = _pad_edges(edge_index[0], edge_index[1], e_pad, N)
    src2, dst2 = _pad_edges(edge_index_2[0], edge_index_2[1], e_pad, N2)
    src3, dst3 = _pad_edges(edge_index_3[0], edge_index_3[1], e_pad, N3)
    sa2, da2 = _pad_edges(assignment_index_2[0], assignment_index_2[1],
                          a2_pad, N2)
    sa3, da3 = _pad_edges(assignment_index_3[0], assignment_index_3[1],
                          a3_pad, N3)
    iota = jnp.arange(N, dtype=jnp.int32)
    sb1, db1 = _pad_edges(iota, batch, nb_pad, B)
    sb2, db2 = _pad_edges(iota, batch_2, nb_pad, B)
    sb3, db3 = _pad_edges(iota, batch_3, nb_pad, B)

    z32 = jnp.zeros((NP // SC_TILES, 32), f32)
    z64 = jnp.zeros((NP // SC_TILES, 64), f32)
    z80 = jnp.zeros((NP // SC_TILES, 80), f32)
    zb80 = jnp.zeros((BP // SC_TILES, 80), f32)

    x_p = jnp.pad(x, ((0, NP - N), (0, 0)))
    iso2_p = jnp.pad(iso_type_2, ((0, NP - N2), (0, 0)))
    iso3_p = jnp.pad(iso_type_3, ((0, NP - N3), (0, 0)))

    agg32 = _sc_segsum(NP, 32, e_pad, NP, 2)
    agg64 = _sc_segsum(NP, 64, e_pad, NP, 2)
    pool_a2 = _sc_segsum(NP, 80, a2_pad, NP, 2)
    pool_a3 = _sc_segsum(NP, 80, a3_pad, NP, 2)
    pool_b = _sc_segsum(NP, 80, nb_pad, BP, 2)

    y1, t1 = _tc_call(
        _two_mm, [_row_spec(D), _wspec((D, 32)), _wspec((D, 32))], 32, 2)(
            x_p, c1_wr, c1_wt)
    s1 = agg32(y1, src1, dst1, z32)
    h1 = _tc_call(
        _combine_elu,
        [_pair_spec(32, 0), _pair_spec(32, 1), _row_spec(32), _wspec((32,))],
        32)(s1, s1, t1, c1_br)

    s2 = agg32(h1, src1, dst1, z32)
    h2 = _tc_call(
        functools.partial(_agg_first, aug=False),
        [_pair_spec(32, 0), _pair_spec(32, 1), _row_spec(32),
         _wspec((32, 64)), _wspec((32, 64)), _wspec((64,))],
        64)(s2, s2, h1, c2_wr, c2_wt, c2_br)

    s3 = agg64(h2, src1, dst1, z64)
    h3a = _tc_call(
        functools.partial(_agg_first, aug=True),
        [_pair_spec(64, 0), _pair_spec(64, 1), _row_spec(64),
         _wspec((64, 64)), _wspec((64, 64)), _wspec((64,))],
        80)(s3, s3, h2, c3_wr, c3_wt, c3_br)

    x1s = pool_b(h3a, sb1, db1, zb80)
    p2s = pool_a2(h3a, sa2, da2, z80)
    p3s = pool_a3(h3a, sa3, da3, z80)

    y4, t4 = _tc_call(
        _pool_transform,
        [_pair_spec(80, 0), _pair_spec(80, 1), _row_spec(NI2),
         _wspec((64, 64)), _wspec((NI2, 64)), _wspec((64, 64)),
         _wspec((NI2, 64))],
        64, 2)(p2s, p2s, iso2_p, c4_wr[:64], c4_wr[64:], c4_wt[:64],
               c4_wt[64:])
    s4 = agg64(y4, src2, dst2, z64)
    g1 = _tc_call(
        _combine_elu,
        [_pair_spec(64, 0), _pair_spec(64, 1), _row_spec(64), _wspec((64,))],
        64)(s4, s4, t4, c4_br)

    s5 = agg64(g1, src2, dst2, z64)
    g2a = _tc_call(
        functools.partial(_agg_first, aug=True),
        [_pair_spec(64, 0), _pair_spec(64, 1), _row_spec(64),
         _wspec((64, 64)), _wspec((64, 64)), _wspec((64,))],
        80)(s5, s5, g1, c5_wr, c5_wt, c5_br)
    x2s = pool_b(g2a, sb2, db2, zb80)

    y6, t6 = _tc_call(
        _pool_transform,
        [_pair_spec(80, 0), _pair_spec(80, 1), _row_spec(NI3),
         _wspec((64, 64)), _wspec((NI3, 64)), _wspec((64, 64)),
         _wspec((NI3, 64))],
        64, 2)(p3s, p3s, iso3_p, c6_wr[:64], c6_wr[64:], c6_wt[:64],
               c6_wt[64:])
    s6 = agg64(y6, src3, dst3, z64)
    m1 = _tc_call(
        _combine_elu,
        [_pair_spec(64, 0), _pair_spec(64, 1), _row_spec(64), _wspec((64,))],
        64)(s6, s6, t6, c6_br)

    s7 = agg64(m1, src3, dst3, z64)
    m2a = _tc_call(
        functools.partial(_agg_first, aug=True),
        [_pair_spec(64, 0), _pair_spec(64, 1), _row_spec(64),
         _wspec((64, 64)), _wspec((64, 64)), _wspec((64,))],
        80)(s7, s7, m1, c7_wr, c7_wt, c7_br)
    x3s = pool_b(m2a, sb3, db3, zb80)

    out = pl.pallas_call(
        _head,
        out_shape=jax.ShapeDtypeStruct((B, NCLS), f32),
    )(x1s, x2s, x3s, fc1_w, fc1_b, fc2_w, fc2_b, fc3_w, fc3_b)
    return out


# speedup vs baseline: 3.3531x; 1.0340x over previous
import functools

import jax
import jax.numpy as jnp
from jax import lax
from jax.experimental import pallas as pl
from jax.experimental.pallas import tpu as pltpu
from jax.experimental.pallas import tpu_sc as plsc

N = 10000
E = 320000
D = 128
N2 = 10000
A2 = 20000
N3 = 10000
A3 = 30000
NI2 = 32
NI3 = 64
NCLS = 10
B = 256

SC_CORES = 2
SC_TILES = 16
NW = SC_CORES * SC_TILES
CH = 64

NP = 10112
BP = 384
RB = 2528


def _round_up(a, m):
    return (a + m - 1) // m * m


@functools.cache
def _sc_segsum(n_src, w, e_pad, n_pad, kg):
    e_tile = e_pad // NW
    n_chunks = e_tile // CH
    ngrp = n_chunks // kg
    assert n_chunks % kg == 0 and ngrp % 2 == 0
    rpt = n_pad // SC_TILES
    spt = n_src // SC_TILES

    mesh = plsc.VectorSubcoreMesh(
        core_axis_name="c", subcore_axis_name="s",
        num_cores=SC_CORES, num_subcores=SC_TILES)

    @functools.partial(
        pl.kernel,
        out_type=jax.ShapeDtypeStruct((SC_CORES, n_pad, w), jnp.float32),
        mesh=mesh,
        compiler_params=pltpu.CompilerParams(use_tc_tiling_on_sc=False),
        scratch_types=[
            pltpu.VMEM_SHARED((n_src, w), jnp.float32),
            pltpu.VMEM_SHARED((n_pad, w), jnp.float32),
            pltpu.VMEM((n_chunks, CH), jnp.int32),
            pltpu.VMEM((n_chunks, CH), jnp.int32),
            pltpu.VMEM((2, kg, CH, w), jnp.float32),
            pltpu.SemaphoreType.DMA,
            pltpu.SemaphoreType.DMA,
            pltpu.SemaphoreType.DMA,
        ],
    )
    def seg(table, src, dst, zeros, out, tbl, acc, idx_s, idx_d, rows,
            sem_g0, sem_g1, sem_s):
        cid = lax.axis_index("c")
        sid = lax.axis_index("s")
        wid = cid * SC_TILES + sid
        sems = (sem_g0, sem_g1)

        pltpu.sync_copy(table.at[pl.ds(sid * spt, spt)],
                        tbl.at[pl.ds(sid * spt, spt)])
        pltpu.sync_copy(zeros, acc.at[pl.ds(sid * rpt, rpt)])
        pltpu.sync_copy(src.at[pl.ds(wid * n_chunks, n_chunks)], idx_s)
        pltpu.sync_copy(dst.at[pl.ds(wid * n_chunks, n_chunks)], idx_d)
        plsc.subcore_barrier()

        def fire(g, s):
            for b in range(kg):
                pltpu.async_copy(tbl.at[idx_s.at[g * kg + b]],
                                 rows.at[s, b], sems[s])

        def drain_scatter(g, s):
            for b in range(kg):
                pltpu.make_async_copy(tbl.at[idx_s.at[0]],
                                      rows.at[s, b], sems[s]).wait()
            ds = [pltpu.async_copy(rows.at[s, b], acc.at[idx_d.at[g * kg + b]],
                                   sem_s, add=True) for b in range(kg)]
            for d in ds:
                d.wait()

        fire(0, 0)

        def grp2(h, carry):
            g0 = h * 2
            fire(g0 + 1, 1)
            drain_scatter(g0, 0)

            @pl.when(g0 + 2 < ngrp)
            def _():
                fire(g0 + 2, 0)

            drain_scatter(g0 + 1, 1)
            return carry

        lax.fori_loop(0, ngrp // 2, grp2, 0)
        plsc.subcore_barrier()
        pltpu.sync_copy(acc.at[pl.ds(sid * rpt, rpt)],
                        out.at[cid, pl.ds(sid * rpt, rpt)])

    return seg


def _pad_edges(src, dst, e_pad, dummy_dst):
    e = src.shape[0]
    src = jnp.pad(src, (0, e_pad - e)).reshape(-1, CH)
    dst = jnp.pad(dst, (0, e_pad - e),
                  constant_values=dummy_dst).reshape(-1, CH)
    return src, dst


def _elu(v):
    return jnp.where(v > 0.0, v, jnp.exp(jnp.minimum(v, 0.0)) - 1.0)


def _row_spec(w):
    return pl.BlockSpec((RB, w), lambda i: (i, 0))


def _pair_spec(w, which):
    return pl.BlockSpec((1, RB, w), lambda i, _w=which: (_w, i, 0))


def _full_spec():
    return pl.BlockSpec(lambda i: None)


def _wspec(shape):
    nd = len(shape)
    return pl.BlockSpec(shape, lambda i: (0,) * nd)


def _tc_aug_call(body, in_specs):
    return pl.pallas_call(
        body, grid=(NP // RB,),
        in_specs=in_specs + [pl.BlockSpec((RB, 1), lambda i: (i, 0))],
        out_specs=(_row_spec(80), pl.BlockSpec((B, 80), lambda i: (0, 0))),
        out_shape=(jax.ShapeDtypeStruct((NP, 80), jnp.float32),
                   jax.ShapeDtypeStruct((B, 80), jnp.float32)))


def _tc_call(body, in_specs, out_w, n_out=1):
    shp = jax.ShapeDtypeStruct((NP, out_w), jnp.float32)
    if n_out == 1:
        out_shape, out_specs = shp, _row_spec(out_w)
    else:
        out_shape = (shp,) * n_out
        out_specs = tuple(_row_spec(out_w) for _ in range(n_out))
    return pl.pallas_call(
        body, grid=(NP // RB,), in_specs=in_specs,
        out_specs=out_specs, out_shape=out_shape)


def _two_mm(x_ref, wr_ref, wt_ref, y_ref, t_ref):
    xv = x_ref[...]
    y_ref[...] = jnp.dot(xv, wr_ref[...], preferred_element_type=jnp.float32)
    t_ref[...] = jnp.dot(xv, wt_ref[...], preferred_element_type=jnp.float32)


def _combine_elu(s_ref0, s_ref1, t_ref, b_ref, o_ref):
    o_ref[...] = _elu(s_ref0[0] + s_ref1[0] + t_ref[...] + b_ref[...])


def _agg_first(s_ref0, s_ref1, h_ref, wr_ref, wt_ref, b_ref, *refs, aug):
    a = s_ref0[0] + s_ref1[0]
    h = h_ref[...]
    v = _elu(jnp.dot(a, wr_ref[...], preferred_element_type=jnp.float32)
             + jnp.dot(h, wt_ref[...], preferred_element_type=jnp.float32)
             + b_ref[...])
    if aug:
        seg_ref, o_ref, pool_ref = refs
        va = jnp.concatenate(
            [v, jnp.ones((v.shape[0], 16), jnp.float32)], axis=1)
        o_ref[...] = va

        @pl.when(pl.program_id(0) == 0)
        def _():
            pool_ref[...] = jnp.zeros_like(pool_ref)

        onehot = (seg_ref[...] == lax.broadcasted_iota(
            jnp.int32, (1, B), 1)).astype(jnp.float32)
        pool_ref[...] += lax.dot_general(
            onehot, va, (((0,), (0,)), ((), ())),
            preferred_element_type=jnp.float32)
    else:
        (o_ref,) = refs
        o_ref[...] = v


def _pool_transform(s_ref0, s_ref1, iso_ref, wrp_ref, wri_ref, wtp_ref,
                    wti_ref, y_ref, t_ref):
    s = s_ref0[0] + s_ref1[0]
    p = s[:, :64] / jnp.maximum(s[:, 64:65], 1.0)
    iso = iso_ref[...]
    f32 = jnp.float32
    y_ref[...] = (jnp.dot(p, wrp_ref[...], preferred_element_type=f32)
                  + jnp.dot(iso, wri_ref[...], preferred_element_type=f32))
    t_ref[...] = (jnp.dot(p, wtp_ref[...], preferred_element_type=f32)
                  + jnp.dot(iso, wti_ref[...], preferred_element_type=f32))


def _head(x1_ref, x2_ref, x3_ref, w1_ref, b1_ref, w2_ref, b2_ref,
          w3_ref, b3_ref, o_ref):
    def pool(ref):
        s = ref[...]
        return s[:, :64] / jnp.maximum(s[:, 64:65], 1.0)

    z = jnp.concatenate([pool(x1_ref), pool(x2_ref), pool(x3_ref)], axis=1)
    f32 = jnp.float32
    z = _elu(jnp.dot(z, w1_ref[...], preferred_element_type=f32) + b1_ref[...])
    z = _elu(jnp.dot(z, w2_ref[...], preferred_element_type=f32) + b2_ref[...])
    z = jnp.dot(z, w3_ref[...], preferred_element_type=f32) + b3_ref[...]
    m = jnp.max(z, axis=1, keepdims=True)
    e = jnp.exp(z - m)
    o_ref[...] = z - m - jnp.log(jnp.sum(e, axis=1, keepdims=True))


def kernel(x, edge_index, batch, assignment_index_2, iso_type_2, edge_index_2,
           batch_2, assignment_index_3, iso_type_3, edge_index_3, batch_3,
           c1_wr, c1_br, c1_wt, c2_wr, c2_br, c2_wt, c3_wr, c3_br, c3_wt,
           c4_wr, c4_br, c4_wt, c5_wr, c5_br, c5_wt, c6_wr, c6_br, c6_wt,
           c7_wr, c7_br, c7_wt, fc1_w, fc1_b, fc2_w, fc2_b, fc3_w, fc3_b):
    f32 = jnp.float32
    e_pad = _round_up(E, NW * CH * 4)
    a2_pad = _round_up(A2, NW * CH * 4)
    a3_pad = _round_up(A3, NW * CH * 4)

    src1, dst1 = _pad_edges(edge_index[0], edge_index[1], e_pad, N)
    src2, dst2 = _pad_edges(edge_index_2[0], edge_index_2[1], e_pad, N2)
    src3, dst3 = _pad_edges(edge_index_3[0], edge_index_3[1], e_pad, N3)
    sa2, da2 = _pad_edges(assignment_index_2[0], assignment_index_2[1],
                          a2_pad, N2)
    sa3, da3 = _pad_edges(assignment_index_3[0], assignment_index_3[1],
                          a3_pad, N3)
    batch_p = jnp.pad(batch, (0, NP - N),
                      constant_values=B).reshape(NP, 1)
    batch2_p = jnp.pad(batch_2, (0, NP - N2),
                       constant_values=B).reshape(NP, 1)
    batch3_p = jnp.pad(batch_3, (0, NP - N3),
                       constant_values=B).reshape(NP, 1)

    z32 = jnp.zeros((NP // SC_TILES, 32), f32)
    z64 = jnp.zeros((NP // SC_TILES, 64), f32)
    z80 = jnp.zeros((NP // SC_TILES, 80), f32)

    x_p = jnp.pad(x, ((0, NP - N), (0, 0)))
    iso2_p = jnp.pad(iso_type_2, ((0, NP - N2), (0, 0)))
    iso3_p = jnp.pad(iso_type_3, ((0, NP - N3), (0, 0)))

    agg32 = _sc_segsum(NP, 32, e_pad, NP, 2)
    agg64 = _sc_segsum(NP, 64, e_pad, NP, 2)
    pool_a2 = _sc_segsum(NP, 80, a2_pad, NP, 2)
    pool_a3 = _sc_segsum(NP, 80, a3_pad, NP, 2)

    y1, t1 = _tc_call(
        _two_mm, [_row_spec(D), _wspec((D, 32)), _wspec((D, 32))], 32, 2)(
            x_p, c1_wr, c1_wt)
    s1 = agg32(y1, src1, dst1, z32)
    h1 = _tc_call(
        _combine_elu,
        [_pair_spec(32, 0), _pair_spec(32, 1), _row_spec(32), _wspec((32,))],
        32)(s1, s1, t1, c1_br)

    s2 = agg32(h1, src1, dst1, z32)
    h2 = _tc_call(
        functools.partial(_agg_first, aug=False),
        [_pair_spec(32, 0), _pair_spec(32, 1), _row_spec(32),
         _wspec((32, 64)), _wspec((32, 64)), _wspec((64,))],
        64)(s2, s2, h1, c2_wr, c2_wt, c2_br)

    s3 = agg64(h2, src1, dst1, z64)
    h3a, x1v = _tc_aug_call(
        functools.partial(_agg_first, aug=True),
        [_pair_spec(64, 0), _pair_spec(64, 1), _row_spec(64),
         _wspec((64, 64)), _wspec((64, 64)), _wspec((64,))])(
            s3, s3, h2, c3_wr, c3_wt, c3_br, batch_p)

    p2s = pool_a2(h3a, sa2, da2, z80)
    p3s = pool_a3(h3a, sa3, da3, z80)

    y4, t4 = _tc_call(
        _pool_transform,
        [_pair_spec(80, 0), _pair_spec(80, 1), _row_spec(NI2),
         _wspec((64, 64)), _wspec((NI2, 64)), _wspec((64, 64)),
         _wspec((NI2, 64))],
        64, 2)(p2s, p2s, iso2_p, c4_wr[:64], c4_wr[64:], c4_wt[:64],
               c4_wt[64:])
    s4 = agg64(y4, src2, dst2, z64)
    g1 = _tc_call(
        _combine_elu,
        [_pair_spec(64, 0), _pair_spec(64, 1), _row_spec(64), _wspec((64,))],
        64)(s4, s4, t4, c4_br)

    s5 = agg64(g1, src2, dst2, z64)
    g2a, x2v = _tc_aug_call(
        functools.partial(_agg_first, aug=True),
        [_pair_spec(64, 0), _pair_spec(64, 1), _row_spec(64),
         _wspec((64, 64)), _wspec((64, 64)), _wspec((64,))])(
            s5, s5, g1, c5_wr, c5_wt, c5_br, batch2_p)

    y6, t6 = _tc_call(
        _pool_transform,
        [_pair_spec(80, 0), _pair_spec(80, 1), _row_spec(NI3),
         _wspec((64, 64)), _wspec((NI3, 64)), _wspec((64, 64)),
         _wspec((NI3, 64))],
        64, 2)(p3s, p3s, iso3_p, c6_wr[:64], c6_wr[64:], c6_wt[:64],
               c6_wt[64:])
    s6 = agg64(y6, src3, dst3, z64)
    m1 = _tc_call(
        _combine_elu,
        [_pair_spec(64, 0), _pair_spec(64, 1), _row_spec(64), _wspec((64,))],
        64)(s6, s6, t6, c6_br)

    s7 = agg64(m1, src3, dst3, z64)
    _, x3v = _tc_aug_call(
        functools.partial(_agg_first, aug=True),
        [_pair_spec(64, 0), _pair_spec(64, 1), _row_spec(64),
         _wspec((64, 64)), _wspec((64, 64)), _wspec((64,))])(
            s7, s7, m1, c7_wr, c7_wt, c7_br, batch3_p)

    out = pl.pallas_call(
        _head,
        out_shape=jax.ShapeDtypeStruct((B, NCLS), f32),
    )(x1v, x2v, x3v, fc1_w, fc1_b, fc2_w, fc2_b, fc3_w, fc3_b)
    return out


# speedup vs baseline: 3.3646x; 1.0034x over previous
import functools

import jax
import jax.numpy as jnp
from jax import lax
from jax.experimental import pallas as pl
from jax.experimental.pallas import tpu as pltpu
from jax.experimental.pallas import tpu_sc as plsc

N = 10000
E = 320000
D = 128
N2 = 10000
A2 = 20000
N3 = 10000
A3 = 30000
NI2 = 32
NI3 = 64
NCLS = 10
B = 256

SC_CORES = 2
SC_TILES = 16
NW = SC_CORES * SC_TILES
CH = 64

NP = 10112
BP = 384
RB = 2528


def _round_up(a, m):
    return (a + m - 1) // m * m


@functools.cache
def _sc_segsum(n_src, w, e_pad, n_pad, kg):
    e_tile = e_pad // NW
    n_chunks = e_tile // CH
    ngrp = n_chunks // kg
    assert n_chunks % kg == 0 and ngrp % 2 == 0
    rpt = n_pad // SC_TILES
    spt = n_src // SC_TILES

    mesh = plsc.VectorSubcoreMesh(
        core_axis_name="c", subcore_axis_name="s",
        num_cores=SC_CORES, num_subcores=SC_TILES)

    @functools.partial(
        pl.kernel,
        out_type=jax.ShapeDtypeStruct((SC_CORES, n_pad, w), jnp.float32),
        mesh=mesh,
        compiler_params=pltpu.CompilerParams(use_tc_tiling_on_sc=False),
        scratch_types=[
            pltpu.VMEM_SHARED((n_src, w), jnp.float32),
            pltpu.VMEM_SHARED((n_pad, w), jnp.float32),
            pltpu.VMEM((n_chunks, CH), jnp.int32),
            pltpu.VMEM((n_chunks, CH), jnp.int32),
            pltpu.VMEM((2, kg, CH, w), jnp.float32),
            pltpu.SemaphoreType.DMA,
            pltpu.SemaphoreType.DMA,
            pltpu.SemaphoreType.DMA,
        ],
    )
    def seg(table, src, dst, zeros, out, tbl, acc, idx_s, idx_d, rows,
            sem_g0, sem_g1, sem_s):
        cid = lax.axis_index("c")
        sid = lax.axis_index("s")
        wid = cid * SC_TILES + sid
        sems = (sem_g0, sem_g1)

        pltpu.sync_copy(table.at[pl.ds(sid * spt, spt)],
                        tbl.at[pl.ds(sid * spt, spt)])
        pltpu.sync_copy(zeros, acc.at[pl.ds(sid * rpt, rpt)])
        pltpu.sync_copy(src.at[pl.ds(wid * n_chunks, n_chunks)], idx_s)
        pltpu.sync_copy(dst.at[pl.ds(wid * n_chunks, n_chunks)], idx_d)
        plsc.subcore_barrier()

        def fire(g, s):
            for b in range(kg):
                pltpu.async_copy(tbl.at[idx_s.at[g * kg + b]],
                                 rows.at[s, b], sems[s])

        def drain_scatter(g, s):
            for b in range(kg):
                pltpu.make_async_copy(tbl.at[idx_s.at[0]],
                                      rows.at[s, b], sems[s]).wait()
            ds = [pltpu.async_copy(rows.at[s, b], acc.at[idx_d.at[g * kg + b]],
                                   sem_s, add=True) for b in range(kg)]
            for d in ds:
                d.wait()

        fire(0, 0)

        def grp2(h, carry):
            g0 = h * 2
            fire(g0 + 1, 1)
            drain_scatter(g0, 0)

            @pl.when(g0 + 2 < ngrp)
            def _():
                fire(g0 + 2, 0)

            drain_scatter(g0 + 1, 1)
            return carry

        lax.fori_loop(0, ngrp // 2, grp2, 0)
        plsc.subcore_barrier()
        pltpu.sync_copy(acc.at[pl.ds(sid * rpt, rpt)],
                        out.at[cid, pl.ds(sid * rpt, rpt)])

    return seg


@functools.cache
def _sc_segsum_pair(n_src, w, e_pad, n_pad):
    e_core = e_pad // SC_TILES
    n_chunks = e_core // CH
    ngrp = n_chunks // 2
    assert n_chunks % 2 == 0 and ngrp % 2 == 0
    kg = 2
    rpt = n_pad // SC_TILES
    spt = n_src // SC_TILES

    mesh = plsc.VectorSubcoreMesh(
        core_axis_name="c", subcore_axis_name="s",
        num_cores=SC_CORES, num_subcores=SC_TILES)

    @functools.partial(
        pl.kernel,
        out_type=jax.ShapeDtypeStruct((SC_CORES, n_pad, w), jnp.float32),
        mesh=mesh,
        compiler_params=pltpu.CompilerParams(use_tc_tiling_on_sc=False),
        scratch_types=[
            pltpu.VMEM_SHARED((n_src, w), jnp.float32),
            pltpu.VMEM_SHARED((n_pad, w), jnp.float32),
            pltpu.VMEM((n_chunks, CH), jnp.int32),
            pltpu.VMEM((n_chunks, CH), jnp.int32),
            pltpu.VMEM((2, kg, CH, w), jnp.float32),
            pltpu.SemaphoreType.DMA,
            pltpu.SemaphoreType.DMA,
            pltpu.SemaphoreType.DMA,
        ],
    )
    def seg(table, src_a, dst_a, src_b, dst_b, zeros, out, tbl, acc,
            idx_s, idx_d, rows, sem_g0, sem_g1, sem_s):
        cid = lax.axis_index("c")
        sid = lax.axis_index("s")
        sems = (sem_g0, sem_g1)

        pltpu.sync_copy(table.at[pl.ds(sid * spt, spt)],
                        tbl.at[pl.ds(sid * spt, spt)])
        pltpu.sync_copy(zeros, acc.at[pl.ds(sid * rpt, rpt)])

        @pl.when(cid == 0)
        def _():
            pltpu.sync_copy(src_a.at[pl.ds(sid * n_chunks, n_chunks)], idx_s)
            pltpu.sync_copy(dst_a.at[pl.ds(sid * n_chunks, n_chunks)], idx_d)

        @pl.when(cid == 1)
        def _():
            pltpu.sync_copy(src_b.at[pl.ds(sid * n_chunks, n_chunks)], idx_s)
            pltpu.sync_copy(dst_b.at[pl.ds(sid * n_chunks, n_chunks)], idx_d)

        plsc.subcore_barrier()

        def fire(g, st):
            for b in range(kg):
                pltpu.async_copy(tbl.at[idx_s.at[g * kg + b]],
                                 rows.at[st, b], sems[st])

        def drain_scatter(g, st):
            for b in range(kg):
                pltpu.make_async_copy(tbl.at[idx_s.at[0]],
                                      rows.at[st, b], sems[st]).wait()
            ds = [pltpu.async_copy(rows.at[st, b],
                                   acc.at[idx_d.at[g * kg + b]],
                                   sem_s, add=True) for b in range(kg)]
            for d in ds:
                d.wait()

        fire(0, 0)

        def grp2(h, carry):
            g0 = h * 2
            fire(g0 + 1, 1)
            drain_scatter(g0, 0)

            @pl.when(g0 + 2 < ngrp)
            def _():
                fire(g0 + 2, 0)

            drain_scatter(g0 + 1, 1)
            return carry

        lax.fori_loop(0, ngrp // 2, grp2, 0)
        plsc.subcore_barrier()
        pltpu.sync_copy(acc.at[pl.ds(sid * rpt, rpt)],
                        out.at[cid, pl.ds(sid * rpt, rpt)])

    return seg


def _pad_edges(src, dst, e_pad, dummy_dst):
    e = src.shape[0]
    src = jnp.pad(src, (0, e_pad - e)).reshape(-1, CH)
    dst = jnp.pad(dst, (0, e_pad - e),
                  constant_values=dummy_dst).reshape(-1, CH)
    return src, dst


def _elu(v):
    return jnp.where(v > 0.0, v, jnp.exp(jnp.minimum(v, 0.0)) - 1.0)


def _row_spec(w):
    return pl.BlockSpec((RB, w), lambda i: (i, 0))


def _pair_spec(w, which):
    return pl.BlockSpec((1, RB, w), lambda i, _w=which: (_w, i, 0))


def _full_spec():
    return pl.BlockSpec(lambda i: None)


def _wspec(shape):
    nd = len(shape)
    return pl.BlockSpec(shape, lambda i: (0,) * nd)


def _tc_aug_call(body, in_specs):
    return pl.pallas_call(
        body, grid=(NP // RB,),
        in_specs=in_specs + [pl.BlockSpec((RB, 1), lambda i: (i, 0))],
        out_specs=(_row_spec(80), pl.BlockSpec((B, 80), lambda i: (0, 0))),
        out_shape=(jax.ShapeDtypeStruct((NP, 80), jnp.float32),
                   jax.ShapeDtypeStruct((B, 80), jnp.float32)))


def _tc_call(body, in_specs, out_w, n_out=1):
    shp = jax.ShapeDtypeStruct((NP, out_w), jnp.float32)
    if n_out == 1:
        out_shape, out_specs = shp, _row_spec(out_w)
    else:
        out_shape = (shp,) * n_out
        out_specs = tuple(_row_spec(out_w) for _ in range(n_out))
    return pl.pallas_call(
        body, grid=(NP // RB,), in_specs=in_specs,
        out_specs=out_specs, out_shape=out_shape)


def _two_mm(x_ref, wr_ref, wt_ref, y_ref, t_ref):
    xv = x_ref[...]
    y_ref[...] = jnp.dot(xv, wr_ref[...], preferred_element_type=jnp.float32)
    t_ref[...] = jnp.dot(xv, wt_ref[...], preferred_element_type=jnp.float32)


def _combine_elu(s_ref0, s_ref1, t_ref, b_ref, o_ref):
    o_ref[...] = _elu(s_ref0[0] + s_ref1[0] + t_ref[...] + b_ref[...])


def _agg_first(s_ref0, s_ref1, h_ref, wr_ref, wt_ref, b_ref, *refs, aug):
    a = s_ref0[0] + s_ref1[0]
    h = h_ref[...]
    v = _elu(jnp.dot(a, wr_ref[...], preferred_element_type=jnp.float32)
             + jnp.dot(h, wt_ref[...], preferred_element_type=jnp.float32)
             + b_ref[...])
    if aug:
        seg_ref, o_ref, pool_ref = refs
        va = jnp.concatenate(
            [v, jnp.ones((v.shape[0], 16), jnp.float32)], axis=1)
        o_ref[...] = va

        @pl.when(pl.program_id(0) == 0)
        def _():
            pool_ref[...] = jnp.zeros_like(pool_ref)

        onehot = (seg_ref[...] == lax.broadcasted_iota(
            jnp.int32, (1, B), 1)).astype(jnp.float32)
        pool_ref[...] += lax.dot_general(
            onehot, va, (((0,), (0,)), ((), ())),
            preferred_element_type=jnp.float32)
    else:
        (o_ref,) = refs
        o_ref[...] = v


def _pool_transform(s_ref, iso_ref, wrp_ref, wri_ref, wtp_ref,
                    wti_ref, y_ref, t_ref):
    s = s_ref[0]
    p = s[:, :64] / jnp.maximum(s[:, 64:65], 1.0)
    iso = iso_ref[...]
    f32 = jnp.float32
    y_ref[...] = (jnp.dot(p, wrp_ref[...], preferred_element_type=f32)
                  + jnp.dot(iso, wri_ref[...], preferred_element_type=f32))
    t_ref[...] = (jnp.dot(p, wtp_ref[...], preferred_element_type=f32)
                  + jnp.dot(iso, wti_ref[...], preferred_element_type=f32))


def _agg_head(s_ref0, s_ref1, h_ref, wr_ref, wt_ref, b_ref,
              x1_ref, x2_ref, w1_ref, b1_ref, w2_ref, b2_ref, w3_ref, b3_ref,
              seg_ref, x3_ref, o_ref):
    f32 = jnp.float32
    a = s_ref0[0] + s_ref1[0]
    h = h_ref[...]
    v = _elu(jnp.dot(a, wr_ref[...], preferred_element_type=f32)
             + jnp.dot(h, wt_ref[...], preferred_element_type=f32)
             + b_ref[...])
    va = jnp.concatenate([v, jnp.ones((v.shape[0], 16), f32)], axis=1)

    i = pl.program_id(0)

    @pl.when(i == 0)
    def _():
        x3_ref[...] = jnp.zeros_like(x3_ref)

    onehot = (seg_ref[...] == lax.broadcasted_iota(
        jnp.int32, (1, B), 1)).astype(f32)
    x3_ref[...] += lax.dot_general(
        onehot, va, (((0,), (0,)), ((), ())), preferred_element_type=f32)

    @pl.when(i == pl.num_programs(0) - 1)
    def _():
        def pool(t):
            return t[:, :64] / jnp.maximum(t[:, 64:65], 1.0)

        z = jnp.concatenate(
            [pool(x1_ref[...]), pool(x2_ref[...]), pool(x3_ref[...])], axis=1)
        z = _elu(jnp.dot(z, w1_ref[...], preferred_element_type=f32)
                 + b1_ref[...])
        z = _elu(jnp.dot(z, w2_ref[...], preferred_element_type=f32)
                 + b2_ref[...])
        z = jnp.dot(z, w3_ref[...], preferred_element_type=f32) + b3_ref[...]
        m = jnp.max(z, axis=1, keepdims=True)
        e = jnp.exp(z - m)
        o_ref[...] = z - m - jnp.log(jnp.sum(e, axis=1, keepdims=True))


def kernel(x, edge_index, batch, assignment_index_2, iso_type_2, edge_index_2,
           batch_2, assignment_index_3, iso_type_3, edge_index_3, batch_3,
           c1_wr, c1_br, c1_wt, c2_wr, c2_br, c2_wt, c3_wr, c3_br, c3_wt,
           c4_wr, c4_br, c4_wt, c5_wr, c5_br, c5_wt, c6_wr, c6_br, c6_wt,
           c7_wr, c7_br, c7_wt, fc1_w, fc1_b, fc2_w, fc2_b, fc3_w, fc3_b):
    f32 = jnp.float32
    e_pad = _round_up(E, NW * CH * 4)
    a2_pad = _round_up(A2, NW * CH * 4)
    a3_pad = _round_up(A3, NW * CH * 4)

    src1, dst1 = _pad_edges(edge_index[0], edge_index[1], e_pad, N)
    src2, dst2 = _pad_edges(edge_index_2[0], edge_index_2[1], e_pad, N2)
    src3, dst3 = _pad_edges(edge_index_3[0], edge_index_3[1], e_pad, N3)
    sa2, da2 = _pad_edges(assignment_index_2[0], assignment_index_2[1],
                          a3_pad, N2)
    sa3, da3 = _pad_edges(assignment_index_3[0], assignment_index_3[1],
                          a3_pad, N3)
    batch_p = jnp.pad(batch, (0, NP - N),
                      constant_values=B).reshape(NP, 1)
    batch2_p = jnp.pad(batch_2, (0, NP - N2),
                       constant_values=B).reshape(NP, 1)
    batch3_p = jnp.pad(batch_3, (0, NP - N3),
                       constant_values=B).reshape(NP, 1)

    z32 = jnp.zeros((NP // SC_TILES, 32), f32)
    z64 = jnp.zeros((NP // SC_TILES, 64), f32)
    z80 = jnp.zeros((NP // SC_TILES, 80), f32)

    x_p = jnp.pad(x, ((0, NP - N), (0, 0)))
    iso2_p = jnp.pad(iso_type_2, ((0, NP - N2), (0, 0)))
    iso3_p = jnp.pad(iso_type_3, ((0, NP - N3), (0, 0)))

    agg32 = _sc_segsum(NP, 32, e_pad, NP, 2)
    agg64 = _sc_segsum(NP, 64, e_pad, NP, 2)
    pool_pair = _sc_segsum_pair(NP, 80, a3_pad, NP)

    y1, t1 = _tc_call(
        _two_mm, [_row_spec(D), _wspec((D, 32)), _wspec((D, 32))], 32, 2)(
            x_p, c1_wr, c1_wt)
    s1 = agg32(y1, src1, dst1, z32)
    h1 = _tc_call(
        _combine_elu,
        [_pair_spec(32, 0), _pair_spec(32, 1), _row_spec(32), _wspec((32,))],
        32)(s1, s1, t1, c1_br)

    s2 = agg32(h1, src1, dst1, z32)
    h2 = _tc_call(
        functools.partial(_agg_first, aug=False),
        [_pair_spec(32, 0), _pair_spec(32, 1), _row_spec(32),
         _wspec((32, 64)), _wspec((32, 64)), _wspec((64,))],
        64)(s2, s2, h1, c2_wr, c2_wt, c2_br)

    s3 = agg64(h2, src1, dst1, z64)
    h3a, x1v = _tc_aug_call(
        functools.partial(_agg_first, aug=True),
        [_pair_spec(64, 0), _pair_spec(64, 1), _row_spec(64),
         _wspec((64, 64)), _wspec((64, 64)), _wspec((64,))])(
            s3, s3, h2, c3_wr, c3_wt, c3_br, batch_p)

    pps = pool_pair(h3a, sa2, da2, sa3, da3, z80)

    y4, t4 = _tc_call(
        _pool_transform,
        [_pair_spec(80, 0), _row_spec(NI2),
         _wspec((64, 64)), _wspec((NI2, 64)), _wspec((64, 64)),
         _wspec((NI2, 64))],
        64, 2)(pps, iso2_p, c4_wr[:64], c4_wr[64:], c4_wt[:64],
               c4_wt[64:])
    s4 = agg64(y4, src2, dst2, z64)
    g1 = _tc_call(
        _combine_elu,
        [_pair_spec(64, 0), _pair_spec(64, 1), _row_spec(64), _wspec((64,))],
        64)(s4, s4, t4, c4_br)

    s5 = agg64(g1, src2, dst2, z64)
    g2a, x2v = _tc_aug_call(
        functools.partial(_agg_first, aug=True),
        [_pair_spec(64, 0), _pair_spec(64, 1), _row_spec(64),
         _wspec((64, 64)), _wspec((64, 64)), _wspec((64,))])(
            s5, s5, g1, c5_wr, c5_wt, c5_br, batch2_p)

    y6, t6 = _tc_call(
        _pool_transform,
        [_pair_spec(80, 1), _row_spec(NI3),
         _wspec((64, 64)), _wspec((NI3, 64)), _wspec((64, 64)),
         _wspec((NI3, 64))],
        64, 2)(pps, iso3_p, c6_wr[:64], c6_wr[64:], c6_wt[:64],
               c6_wt[64:])
    s6 = agg64(y6, src3, dst3, z64)
    m1 = _tc_call(
        _combine_elu,
        [_pair_spec(64, 0), _pair_spec(64, 1), _row_spec(64), _wspec((64,))],
        64)(s6, s6, t6, c6_br)

    s7 = agg64(m1, src3, dst3, z64)
    _, out = pl.pallas_call(
        _agg_head, grid=(NP // RB,),
        in_specs=[_pair_spec(64, 0), _pair_spec(64, 1), _row_spec(64),
                  _wspec((64, 64)), _wspec((64, 64)), _wspec((64,)),
                  _wspec((B, 80)), _wspec((B, 80)),
                  _wspec((3 * 64, 64)), _wspec((64,)),
                  _wspec((64, 32)), _wspec((32,)),
                  _wspec((32, NCLS)), _wspec((NCLS,)),
                  pl.BlockSpec((RB, 1), lambda i: (i, 0))],
        out_specs=(pl.BlockSpec((B, 80), lambda i: (0, 0)),
                   pl.BlockSpec((B, NCLS), lambda i: (0, 0))),
        out_shape=(jax.ShapeDtypeStruct((B, 80), f32),
                   jax.ShapeDtypeStruct((B, NCLS), f32)),
    )(s7, s7, m1, c7_wr, c7_wt, c7_br, x1v, x2v,
      fc1_w, fc1_b, fc2_w, fc2_b, fc3_w, fc3_b, batch3_p)
    return out


# speedup vs baseline: 3.3713x; 1.0020x over previous
import functools

import jax
import jax.numpy as jnp
from jax import lax
from jax.experimental import pallas as pl
from jax.experimental.pallas import tpu as pltpu
from jax.experimental.pallas import tpu_sc as plsc

N = 10000
E = 320000
D = 128
N2 = 10000
A2 = 20000
N3 = 10000
A3 = 30000
NI2 = 32
NI3 = 64
NCLS = 10
B = 256

SC_CORES = 2
SC_TILES = 16
NW = SC_CORES * SC_TILES
CH = 64

NP = 10112
RB = 2528


def _round_up(a, m):
    return (a + m - 1) // m * m


@functools.cache
def _sc_segsum(n_src, w, e_pad, n_pad, kg):
    e_tile = e_pad // NW
    n_chunks = e_tile // CH
    ngrp = n_chunks // kg
    assert n_chunks % kg == 0 and ngrp % 2 == 0
    rpt = n_pad // SC_TILES
    spt = n_src // SC_TILES

    mesh = plsc.VectorSubcoreMesh(
        core_axis_name="c", subcore_axis_name="s",
        num_cores=SC_CORES, num_subcores=SC_TILES)

    @functools.partial(
        pl.kernel,
        out_type=jax.ShapeDtypeStruct((SC_CORES, n_pad, w), jnp.float32),
        mesh=mesh,
        compiler_params=pltpu.CompilerParams(use_tc_tiling_on_sc=False),
        scratch_types=[
            pltpu.VMEM_SHARED((n_src, w), jnp.float32),
            pltpu.VMEM_SHARED((n_pad, w), jnp.float32),
            pltpu.VMEM((n_chunks, CH), jnp.int32),
            pltpu.VMEM((n_chunks, CH), jnp.int32),
            pltpu.VMEM((2, kg, CH, w), jnp.float32),
            pltpu.SemaphoreType.DMA,
            pltpu.SemaphoreType.DMA,
            pltpu.SemaphoreType.DMA,
        ],
    )
    def seg(table, src, dst, zeros, out, tbl, acc, idx_s, idx_d, rows,
            sem_g0, sem_g1, sem_s):
        cid = lax.axis_index("c")
        sid = lax.axis_index("s")
        wid = cid * SC_TILES + sid
        sems = (sem_g0, sem_g1)

        pltpu.sync_copy(table.at[pl.ds(sid * spt, spt)],
                        tbl.at[pl.ds(sid * spt, spt)])
        pltpu.sync_copy(zeros, acc.at[pl.ds(sid * rpt, rpt)])
        pltpu.sync_copy(src.at[pl.ds(wid * n_chunks, n_chunks)], idx_s)
        pltpu.sync_copy(dst.at[pl.ds(wid * n_chunks, n_chunks)], idx_d)
        plsc.subcore_barrier()

        def fire(g, s):
            for b in range(kg):
                pltpu.async_copy(tbl.at[idx_s.at[g * kg + b]],
                                 rows.at[s, b], sems[s])

        def drain_scatter(g, s):
            for b in range(kg):
                pltpu.make_async_copy(tbl.at[idx_s.at[0]],
                                      rows.at[s, b], sems[s]).wait()
            ds = [pltpu.async_copy(rows.at[s, b], acc.at[idx_d.at[g * kg + b]],
                                   sem_s, add=True) for b in range(kg)]
            for d in ds:
                d.wait()

        fire(0, 0)

        def grp2(h, carry):
            g0 = h * 2
            fire(g0 + 1, 1)
            drain_scatter(g0, 0)

            @pl.when(g0 + 2 < ngrp)
            def _():
                fire(g0 + 2, 0)

            drain_scatter(g0 + 1, 1)
            return carry

        lax.fori_loop(0, ngrp // 2, grp2, 0)
        plsc.subcore_barrier()
        pltpu.sync_copy(acc.at[pl.ds(sid * rpt, rpt)],
                        out.at[cid, pl.ds(sid * rpt, rpt)])

    return seg


@functools.cache
def _sc_segsum_pair(n_src, w, e_pad, n_pad):
    e_core = e_pad // SC_TILES
    n_chunks = e_core // CH
    ngrp = n_chunks // 2
    assert n_chunks % 2 == 0 and ngrp % 2 == 0
    kg = 2
    rpt = n_pad // SC_TILES
    spt = n_src // SC_TILES

    mesh = plsc.VectorSubcoreMesh(
        core_axis_name="c", subcore_axis_name="s",
        num_cores=SC_CORES, num_subcores=SC_TILES)

    @functools.partial(
        pl.kernel,
        out_type=jax.ShapeDtypeStruct((SC_CORES, n_pad, w), jnp.float32),
        mesh=mesh,
        compiler_params=pltpu.CompilerParams(use_tc_tiling_on_sc=False),
        scratch_types=[
            pltpu.VMEM_SHARED((n_src, w), jnp.float32),
            pltpu.VMEM_SHARED((n_pad, w), jnp.float32),
            pltpu.VMEM((n_chunks, CH), jnp.int32),
            pltpu.VMEM((n_chunks, CH), jnp.int32),
            pltpu.VMEM((2, kg, CH, w), jnp.float32),
            pltpu.SemaphoreType.DMA,
            pltpu.SemaphoreType.DMA,
            pltpu.SemaphoreType.DMA,
        ],
    )
    def seg(table, src_a, dst_a, src_b, dst_b, zeros, out, tbl, acc,
            idx_s, idx_d, rows, sem_g0, sem_g1, sem_s):
        cid = lax.axis_index("c")
        sid = lax.axis_index("s")
        sems = (sem_g0, sem_g1)

        pltpu.sync_copy(table.at[pl.ds(sid * spt, spt)],
                        tbl.at[pl.ds(sid * spt, spt)])
        pltpu.sync_copy(zeros, acc.at[pl.ds(sid * rpt, rpt)])

        @pl.when(cid == 0)
        def _():
            pltpu.sync_copy(src_a.at[pl.ds(sid * n_chunks, n_chunks)], idx_s)
            pltpu.sync_copy(dst_a.at[pl.ds(sid * n_chunks, n_chunks)], idx_d)

        @pl.when(cid == 1)
        def _():
            pltpu.sync_copy(src_b.at[pl.ds(sid * n_chunks, n_chunks)], idx_s)
            pltpu.sync_copy(dst_b.at[pl.ds(sid * n_chunks, n_chunks)], idx_d)

        plsc.subcore_barrier()

        def fire(g, st):
            for b in range(kg):
                pltpu.async_copy(tbl.at[idx_s.at[g * kg + b]],
                                 rows.at[st, b], sems[st])

        def drain_scatter(g, st):
            for b in range(kg):
                pltpu.make_async_copy(tbl.at[idx_s.at[0]],
                                      rows.at[st, b], sems[st]).wait()
            ds = [pltpu.async_copy(rows.at[st, b],
                                   acc.at[idx_d.at[g * kg + b]],
                                   sem_s, add=True) for b in range(kg)]
            for d in ds:
                d.wait()

        fire(0, 0)

        def grp2(h, carry):
            g0 = h * 2
            fire(g0 + 1, 1)
            drain_scatter(g0, 0)

            @pl.when(g0 + 2 < ngrp)
            def _():
                fire(g0 + 2, 0)

            drain_scatter(g0 + 1, 1)
            return carry

        lax.fori_loop(0, ngrp // 2, grp2, 0)
        plsc.subcore_barrier()
        pltpu.sync_copy(acc.at[pl.ds(sid * rpt, rpt)],
                        out.at[cid, pl.ds(sid * rpt, rpt)])

    return seg


def _pad_edges(src, dst, e_pad, dummy_dst):
    e = src.shape[0]
    src = jnp.pad(src, (0, e_pad - e)).reshape(-1, CH)
    dst = jnp.pad(dst, (0, e_pad - e),
                  constant_values=dummy_dst).reshape(-1, CH)
    return src, dst


def _elu(v):
    return jnp.where(v > 0.0, v, jnp.exp(jnp.minimum(v, 0.0)) - 1.0)


def _row_spec(w):
    return pl.BlockSpec((RB, w), lambda i: (i, 0))


def _pair_spec(w, which):
    return pl.BlockSpec((1, RB, w), lambda i, _w=which: (_w, i, 0))


def _wspec(shape):
    nd = len(shape)
    return pl.BlockSpec(shape, lambda i: (0,) * nd)


def _tc_aug_call(body, in_specs):
    return pl.pallas_call(
        body, grid=(NP // RB,),
        in_specs=in_specs + [pl.BlockSpec((RB, 1), lambda i: (i, 0))],
        out_specs=(_row_spec(80), pl.BlockSpec((B, 80), lambda i: (0, 0))),
        out_shape=(jax.ShapeDtypeStruct((NP, 80), jnp.float32),
                   jax.ShapeDtypeStruct((B, 80), jnp.float32)))


def _tc_call(body, in_specs, out_w, n_out=1):
    shp = jax.ShapeDtypeStruct((NP, out_w), jnp.float32)
    if n_out == 1:
        out_shape, out_specs = shp, _row_spec(out_w)
    else:
        out_shape = (shp,) * n_out
        out_specs = tuple(_row_spec(out_w) for _ in range(n_out))
    return pl.pallas_call(
        body, grid=(NP // RB,), in_specs=in_specs,
        out_specs=out_specs, out_shape=out_shape)


def _two_mm(x_ref, wr_ref, wt_ref, y_ref, t_ref):
    xv = x_ref[...]
    y_ref[...] = jnp.dot(xv, wr_ref[...], preferred_element_type=jnp.float32)
    t_ref[...] = jnp.dot(xv, wt_ref[...], preferred_element_type=jnp.float32)


def _combine_elu(s_ref0, s_ref1, t_ref, b_ref, o_ref):
    o_ref[...] = _elu(s_ref0[0] + s_ref1[0] + t_ref[...] + b_ref[...])


def _agg_first(s_ref0, s_ref1, h_ref, wr_ref, wt_ref, b_ref, *refs, aug):
    a = s_ref0[0] + s_ref1[0]
    h = h_ref[...]
    v = _elu(jnp.dot(a, wr_ref[...], preferred_element_type=jnp.float32)
             + jnp.dot(h, wt_ref[...], preferred_element_type=jnp.float32)
             + b_ref[...])
    if aug:
        seg_ref, o_ref, pool_ref = refs
        va = jnp.concatenate(
            [v, jnp.ones((v.shape[0], 16), jnp.float32)], axis=1)
        o_ref[...] = va

        @pl.when(pl.program_id(0) == 0)
        def _():
            pool_ref[...] = jnp.zeros_like(pool_ref)

        onehot = (seg_ref[...] == lax.broadcasted_iota(
            jnp.int32, (1, B), 1)).astype(jnp.float32)
        pool_ref[...] += lax.dot_general(
            onehot, va, (((0,), (0,)), ((), ())),
            preferred_element_type=jnp.float32)
    else:
        (o_ref,) = refs
        o_ref[...] = v


def _pool_transform(s_ref, iso_ref, wrp_ref, wri_ref, wtp_ref,
                    wti_ref, y_ref, t_ref):
    s = s_ref[0]
    p = s[:, :64] / jnp.maximum(s[:, 64:65], 1.0)
    iso = iso_ref[...]
    f32 = jnp.float32
    y_ref[...] = (jnp.dot(p, wrp_ref[...], preferred_element_type=f32)
                  + jnp.dot(iso, wri_ref[...], preferred_element_type=f32))
    t_ref[...] = (jnp.dot(p, wtp_ref[...], preferred_element_type=f32)
                  + jnp.dot(iso, wti_ref[...], preferred_element_type=f32))


def _agg_head(s_ref0, s_ref1, h_ref, wr_ref, wt_ref, b_ref,
              x1_ref, x2_ref, w1_ref, b1_ref, w2_ref, b2_ref, w3_ref, b3_ref,
              seg_ref, x3_ref, o_ref):
    f32 = jnp.float32
    a = s_ref0[0] + s_ref1[0]
    h = h_ref[...]
    v = _elu(jnp.dot(a, wr_ref[...], preferred_element_type=f32)
             + jnp.dot(h, wt_ref[...], preferred_element_type=f32)
             + b_ref[...])
    va = jnp.concatenate([v, jnp.ones((v.shape[0], 16), f32)], axis=1)

    i = pl.program_id(0)

    @pl.when(i == 0)
    def _():
        x3_ref[...] = jnp.zeros_like(x3_ref)

    onehot = (seg_ref[...] == lax.broadcasted_iota(
        jnp.int32, (1, B), 1)).astype(f32)
    x3_ref[...] += lax.dot_general(
        onehot, va, (((0,), (0,)), ((), ())), preferred_element_type=f32)

    @pl.when(i == pl.num_programs(0) - 1)
    def _():
        def pool(t):
            return t[:, :64] / jnp.maximum(t[:, 64:65], 1.0)

        z = jnp.concatenate(
            [pool(x1_ref[...]), pool(x2_ref[...]), pool(x3_ref[...])], axis=1)
        z = _elu(jnp.dot(z, w1_ref[...], preferred_element_type=f32)
                 + b1_ref[...])
        z = _elu(jnp.dot(z, w2_ref[...], preferred_element_type=f32)
                 + b2_ref[...])
        z = jnp.dot(z, w3_ref[...], preferred_element_type=f32) + b3_ref[...]
        m = jnp.max(z, axis=1, keepdims=True)
        e = jnp.exp(z - m)
        o_ref[...] = z - m - jnp.log(jnp.sum(e, axis=1, keepdims=True))


def kernel(x, edge_index, batch, assignment_index_2, iso_type_2, edge_index_2,
           batch_2, assignment_index_3, iso_type_3, edge_index_3, batch_3,
           c1_wr, c1_br, c1_wt, c2_wr, c2_br, c2_wt, c3_wr, c3_br, c3_wt,
           c4_wr, c4_br, c4_wt, c5_wr, c5_br, c5_wt, c6_wr, c6_br, c6_wt,
           c7_wr, c7_br, c7_wt, fc1_w, fc1_b, fc2_w, fc2_b, fc3_w, fc3_b):
    f32 = jnp.float32
    e_pad = _round_up(E, NW * CH * 4)
    a3_pad = _round_up(A3, NW * CH * 4)

    src1, dst1 = _pad_edges(edge_index[0], edge_index[1], e_pad, N)
    src2, dst2 = _pad_edges(edge_index_2[0], edge_index_2[1], e_pad, N2)
    src3, dst3 = _pad_edges(edge_index_3[0], edge_index_3[1], e_pad, N3)
    sa2, da2 = _pad_edges(assignment_index_2[0], assignment_index_2[1],
                          a3_pad, N2)
    sa3, da3 = _pad_edges(assignment_index_3[0], assignment_index_3[1],
                          a3_pad, N3)
    batch_p = jnp.pad(batch, (0, NP - N),
                      constant_values=B).reshape(NP, 1)
    batch2_p = jnp.pad(batch_2, (0, NP - N2),
                       constant_values=B).reshape(NP, 1)
    batch3_p = jnp.pad(batch_3, (0, NP - N3),
                       constant_values=B).reshape(NP, 1)

    z32 = jnp.zeros((NP // SC_TILES, 32), f32)
    z64 = jnp.zeros((NP // SC_TILES, 64), f32)
    z80 = jnp.zeros((NP // SC_TILES, 80), f32)

    x_p = jnp.pad(x, ((0, NP - N), (0, 0)))
    iso2_p = jnp.pad(iso_type_2, ((0, NP - N2), (0, 0)))
    iso3_p = jnp.pad(iso_type_3, ((0, NP - N3), (0, 0)))

    agg32 = _sc_segsum(NP, 32, e_pad, NP, 2)
    agg64 = _sc_segsum(NP, 64, e_pad, NP, 2)
    pool_pair = _sc_segsum_pair(NP, 80, a3_pad, NP)

    y1, t1 = _tc_call(
        _two_mm, [_row_spec(D), _wspec((D, 32)), _wspec((D, 32))], 32, 2)(
            x_p, c1_wr, c1_wt)
    s1 = agg32(y1, src1, dst1, z32)
    h1 = _tc_call(
        _combine_elu,
        [_pair_spec(32, 0), _pair_spec(32, 1), _row_spec(32), _wspec((32,))],
        32)(s1, s1, t1, c1_br)

    s2 = agg32(h1, src1, dst1, z32)
    h2 = _tc_call(
        functools.partial(_agg_first, aug=False),
        [_pair_spec(32, 0), _pair_spec(32, 1), _row_spec(32),
         _wspec((32, 64)), _wspec((32, 64)), _wspec((64,))],
        64)(s2, s2, h1, c2_wr, c2_wt, c2_br)

    s3 = agg64(h2, src1, dst1, z64)
    h3a, x1v = _tc_aug_call(
        functools.partial(_agg_first, aug=True),
        [_pair_spec(64, 0), _pair_spec(64, 1), _row_spec(64),
         _wspec((64, 64)), _wspec((64, 64)), _wspec((64,))])(
            s3, s3, h2, c3_wr, c3_wt, c3_br, batch_p)

    pps = pool_pair(h3a, sa2, da2, sa3, da3, z80)

    y4, t4 = _tc_call(
        _pool_transform,
        [_pair_spec(80, 0), _row_spec(NI2),
         _wspec((64, 64)), _wspec((NI2, 64)), _wspec((64, 64)),
         _wspec((NI2, 64))],
        64, 2)(pps, iso2_p, c4_wr[:64], c4_wr[64:], c4_wt[:64],
               c4_wt[64:])
    s4 = agg64(y4, src2, dst2, z64)
    g1 = _tc_call(
        _combine_elu,
        [_pair_spec(64, 0), _pair_spec(64, 1), _row_spec(64), _wspec((64,))],
        64)(s4, s4, t4, c4_br)

    s5 = agg64(g1, src2, dst2, z64)
    g2a, x2v = _tc_aug_call(
        functools.partial(_agg_first, aug=True),
        [_pair_spec(64, 0), _pair_spec(64, 1), _row_spec(64),
         _wspec((64, 64)), _wspec((64, 64)), _wspec((64,))])(
            s5, s5, g1, c5_wr, c5_wt, c5_br, batch2_p)

    y6, t6 = _tc_call(
        _pool_transform,
        [_pair_spec(80, 1), _row_spec(NI3),
         _wspec((64, 64)), _wspec((NI3, 64)), _wspec((64, 64)),
         _wspec((NI3, 64))],
        64, 2)(pps, iso3_p, c6_wr[:64], c6_wr[64:], c6_wt[:64],
               c6_wt[64:])
    s6 = agg64(y6, src3, dst3, z64)
    m1 = _tc_call(
        _combine_elu,
        [_pair_spec(64, 0), _pair_spec(64, 1), _row_spec(64), _wspec((64,))],
        64)(s6, s6, t6, c6_br)

    s7 = agg64(m1, src3, dst3, z64)
    _, out = pl.pallas_call(
        _agg_head, grid=(NP // RB,),
        in_specs=[_pair_spec(64, 0), _pair_spec(64, 1), _row_spec(64),
                  _wspec((64, 64)), _wspec((64, 64)), _wspec((64,)),
                  _wspec((B, 80)), _wspec((B, 80)),
                  _wspec((3 * 64, 64)), _wspec((64,)),
                  _wspec((64, 32)), _wspec((32,)),
                  _wspec((32, NCLS)), _wspec((NCLS,)),
                  pl.BlockSpec((RB, 1), lambda i: (i, 0))],
        out_specs=(pl.BlockSpec((B, 80), lambda i: (0, 0)),
                   pl.BlockSpec((B, NCLS), lambda i: (0, 0))),
        out_shape=(jax.ShapeDtypeStruct((B, 80), f32),
                   jax.ShapeDtypeStruct((B, NCLS), f32)),
    )(s7, s7, m1, c7_wr, c7_wt, c7_br, x1v, x2v,
      fc1_w, fc1_b, fc2_w, fc2_b, fc3_w, fc3_b, batch3_p)
    return out
